# baseline (speedup 1.0000x reference)
import jax
import jax.numpy as jnp
from jax.experimental import pallas as pl
from jax.experimental.pallas import tpu as pltpu

_N, _P, _D = 4, 4096, 3
_QC = 1024
_NQ = _P // _QC


def _chamfer_kernel(src_ref, tgt_ref, out_ref, L_ref):
    b = pl.program_id(0)
    j = pl.program_id(1)

    @pl.when(j == 0)
    def _():
        T = tgt_ref[0]
        y2 = jnp.sum(T * T, axis=1, keepdims=True)
        y2_hi = y2.astype(jnp.bfloat16).astype(jnp.float32)
        y2_lo = y2 - y2_hi
        L_ref[...] = jnp.concatenate([T, y2_hi, y2_lo],
                                     axis=1)

    Sc = src_ref[0]
    Sc8 = jnp.concatenate(
        [Sc, jnp.zeros((_QC, 8 - _D), jnp.float32)], axis=1)
    St = jnp.swapaxes(Sc8, 0, 1)[:_D]

    x2 = jnp.sum(St * St, axis=0, keepdims=True)
    ones_q = jnp.ones((1, _QC), jnp.float32)
    R = jnp.concatenate([-2.0 * St, ones_q, ones_q],
                        axis=0)

    d = jax.lax.dot_general(
        L_ref[...], R, (((1,), (0,)), ((), ())),
        preferred_element_type=jnp.float32,
    )
    m = jnp.min(d, axis=0, keepdims=True)

    s = jnp.sum(m + x2, keepdims=True) * (1.0 / _N)

    @pl.when(jnp.logical_and(b == 0, j == 0))
    def _():
        out_ref[...] = jnp.zeros_like(out_ref)

    out_ref[...] += s


def kernel(source_cloud, target_cloud):
    out = pl.pallas_call(
        _chamfer_kernel,
        grid=(_N, _NQ),
        in_specs=[
            pl.BlockSpec((1, _QC, _D), lambda b, j: (b, j, 0)),
            pl.BlockSpec((1, _P, _D), lambda b, j: (b, 0, 0)),
        ],
        out_specs=pl.BlockSpec((1, 1), lambda b, j: (0, 0)),
        out_shape=jax.ShapeDtypeStruct((1, 1), jnp.float32),
        scratch_shapes=[pltpu.VMEM((_P, 5), jnp.float32)],
    )(source_cloud, target_cloud)
    return out[0, 0]
